# baseline (speedup 1.0000x reference)
import functools

import jax
import jax.numpy as jnp
from jax import lax
from jax.experimental import pallas as pl
from jax.experimental.pallas import tpu as pltpu
from jax.experimental.pallas import tpu_sc as plsc

B, L, D = 64, 12, 32768
K = 64
BATCH_K = K * B * L
N = B * L * D

NC, NS = 2, 16
NW = NC * NS
PER_W = N // NW
CHUNK = 16384
NCH = PER_W // CHUNK
NBINS = 65536
UNROLL = 12

_MESH = plsc.VectorSubcoreMesh(
    core_axis_name="c", subcore_axis_name="s", num_cores=NC, num_subcores=NS
)


def _monotonic_key(v):
  s = plsc.bitcast(v, jnp.int32)
  m = lax.shift_right_arithmetic(s, 31)
  return lax.bitwise_xor(s, lax.bitwise_and(m, jnp.int32(0x7FFFFFFF)))


def _hist_body(feat_ref, sel_ref, out_ref, buf0, buf1, hist, bvec, sem0, sem1,
               *, low_bits):
  wid = lax.axis_index("s") * NC + lax.axis_index("c")

  def _src(ch):
    b_off, rem = divmod(ch, 2 * L)
    lay, half = divmod(rem, 2)
    return feat_ref.at[2 * wid + b_off, lay, pl.ds(half * CHUNK, CHUNK)]

  zero16 = jnp.zeros((16,), jnp.int32)

  def zbody(i, c):
    hist[pl.ds(i * 16, 16)] = zero16
    return c

  lax.fori_loop(0, NBINS // 16, zbody, 0)

  if low_bits:
    pltpu.sync_copy(sel_ref.at[pl.ds(0, 16)], bvec)
    bval = bvec[...]

  bufs = (buf0, buf1)
  sems = (sem0, sem1)
  cps = [None, None]
  cps[0] = pltpu.async_copy(_src(0), buf0, sem0)
  for ch in range(NCH):
    if ch + 1 < NCH:
      cps[(ch + 1) % 2] = pltpu.async_copy(
          _src(ch + 1), bufs[(ch + 1) % 2], sems[(ch + 1) % 2])
    cps[ch % 2].wait()
    buf = bufs[ch % 2]

    @plsc.parallel_loop(0, CHUNK // 16, unroll=UNROLL)
    def _(j):
      v = buf[pl.ds(j * 16, 16)]
      key = _monotonic_key(v)
      hi = lax.shift_right_arithmetic(key, 16) + jnp.int32(32768)
      if low_bits:
        binv = lax.bitwise_and(key, jnp.int32(0xFFFF))
        elig = hi == bval
        cnt, last = plsc.scan_count(binv, elig)
      else:
        binv = hi
        cnt, last = plsc.scan_count(binv)
      plsc.addupdate_scatter(hist, [binv], cnt, mask=last)

  pltpu.sync_copy(hist, out_ref.at[wid])


def _make_hist(low_bits):
  body = functools.partial(_hist_body, low_bits=low_bits)
  return pl.kernel(
      body,
      out_type=jax.ShapeDtypeStruct((NW, NBINS), jnp.int32),
      mesh=_MESH,
      compiler_params=pltpu.CompilerParams(needs_layout_passes=False),
      scratch_types=[
          pltpu.VMEM((CHUNK,), jnp.float32),
          pltpu.VMEM((CHUNK,), jnp.float32),
          pltpu.VMEM((NBINS,), jnp.int32),
          pltpu.VMEM((16,), jnp.int32),
          pltpu.SemaphoreType.DMA,
          pltpu.SemaphoreType.DMA,
      ],
  )


_hist_hi = _make_hist(False)
_hist_lo = _make_hist(True)


def _lin_idx():
  return (lax.broadcasted_iota(jnp.int32, (512, 128), 0) * 128
          + lax.broadcasted_iota(jnp.int32, (512, 128), 1))


def _find_bucket(tot, k):
  lin = _lin_idx()

  def step(_, lohi):
    lo, hi = lohi
    mid = lax.div(lo + hi, jnp.int32(2))
    cnt = jnp.sum(jnp.where(lin > mid, tot, 0))
    ok = cnt < k
    return jnp.where(ok, lo, mid + 1), jnp.where(ok, mid, hi)

  b, _ = lax.fori_loop(0, 16, step, (jnp.int32(0), jnp.int32(NBINS - 1)))
  n_above = jnp.sum(jnp.where(lin > b, tot, 0))
  return b, n_above


def _select1_body(h_ref, out_ref):
  h = h_ref[...]
  tot = jnp.sum(h, axis=0)
  b, n_above = _find_bucket(tot, jnp.int32(BATCH_K))
  r = jnp.int32(BATCH_K) - n_above
  ri = lax.broadcasted_iota(jnp.int32, (8, 128), 0)
  out_ref[...] = jnp.where(ri == 0, b, r)


def _select1(hist3d):
  return pl.pallas_call(
      _select1_body,
      out_shape=jax.ShapeDtypeStruct((8, 128), jnp.int32),
  )(hist3d)


MB_B = 2
_BLIDX = MB_B * L * D


def _mask_body(x_ref, h_ref, sel_ref, o_ref, st_ref, cnt_ref):
  i = pl.program_id(0)

  @pl.when(i == 0)
  def _():
    cnt_ref[0] = 0
    tot = jnp.sum(h_ref[...], axis=0)
    selv = sel_ref[...]
    ri = lax.broadcasted_iota(jnp.int32, (8, 128), 0)
    neg = jnp.int32(-(1 << 30))
    b = jnp.max(jnp.where(ri == 0, selv, neg))
    r = jnp.max(jnp.where(ri == 1, selv, neg))
    tlow, e_at = _find_bucket(tot, r)
    st_ref[0] = lax.shift_left(b - jnp.int32(32768), 16) + tlow
    st_ref[1] = r - e_at

  x = x_ref[...]
  s = lax.bitcast_convert_type(x, jnp.int32)
  m = lax.shift_right_arithmetic(s, 31)
  key = lax.bitwise_xor(s, lax.bitwise_and(m, jnp.int32(0x7FFFFFFF)))
  t = st_ref[0]
  q = st_ref[1]
  gt = key > t
  eq = key == t
  relu = jnp.maximum(x, 0.0)
  zero = jnp.float32(0.0)
  n_eq = jnp.sum(eq.astype(jnp.int32))
  need = q - cnt_ref[0]

  o_ref[...] = jnp.where(gt, relu, zero)

  @pl.when(jnp.logical_and(need > 0, n_eq > 0))
  def _():
    @pl.when(need >= n_eq)
    def _():
      o_ref[...] = jnp.where(jnp.logical_or(gt, eq), relu, zero)

    @pl.when(need < n_eq)
    def _():
      lidx = ((lax.broadcasted_iota(jnp.int32, (MB_B, L, D), 0) * L
               + lax.broadcasted_iota(jnp.int32, (MB_B, L, D), 1)) * D
              + lax.broadcasted_iota(jnp.int32, (MB_B, L, D), 2))

      def step(_, lohi):
        lo, hi = lohi
        mid = lax.div(lo + hi, jnp.int32(2))
        cnt = jnp.sum(jnp.logical_and(eq, lidx <= mid).astype(jnp.int32))
        ok = cnt >= need
        return jnp.where(ok, lo, mid + 1), jnp.where(ok, mid, hi)

      lo, _hi = lax.fori_loop(0, 20, step, (jnp.int32(0),
                                            jnp.int32(_BLIDX - 1)))
      sel = jnp.logical_or(gt, jnp.logical_and(eq, lidx <= lo))
      o_ref[...] = jnp.where(sel, relu, zero)

  cnt_ref[0] = cnt_ref[0] + n_eq


def _mask(x3d, hist3d, sel1):
  return pl.pallas_call(
      _mask_body,
      grid=(B // MB_B,),
      in_specs=[
          pl.BlockSpec((MB_B, L, D), lambda i: (i, 0, 0)),
          pl.BlockSpec((32, 512, 128), lambda i: (0, 0, 0)),
          pl.BlockSpec((8, 128), lambda i: (0, 0)),
      ],
      out_specs=pl.BlockSpec((MB_B, L, D), lambda i: (i, 0, 0)),
      out_shape=jax.ShapeDtypeStruct((B, L, D), jnp.float32),
      scratch_shapes=[pltpu.SMEM((2,), jnp.int32),
                      pltpu.SMEM((1,), jnp.int32)],
  )(x3d, hist3d, sel1)


@jax.jit
def kernel(features):
  hist1 = _hist_hi(features, jnp.zeros((1024,), jnp.int32))
  sel1 = _select1(hist1.reshape(32, 512, 128))
  hist2 = _hist_lo(features, sel1.reshape(-1))
  return _mask(features, hist2.reshape(32, 512, 128), sel1)


# speedup vs baseline: 1.1570x; 1.1570x over previous
import functools

import jax
import jax.numpy as jnp
from jax import lax
from jax.experimental import pallas as pl
from jax.experimental.pallas import tpu as pltpu
from jax.experimental.pallas import tpu_sc as plsc

B, L, D = 64, 12, 32768
K = 64
BATCH_K = K * B * L
N = B * L * D

NC, NS = 2, 16
NW = NC * NS
PER_W = N // NW
CHUNK = 16384
NCH = PER_W // CHUNK
NBINS = 65536
UNROLL = 8

_MESH = plsc.VectorSubcoreMesh(
    core_axis_name="c", subcore_axis_name="s", num_cores=NC, num_subcores=NS
)


def _monotonic_key(v):
  s = plsc.bitcast(v, jnp.int32)
  m = lax.shift_right_arithmetic(s, 31)
  return lax.bitwise_xor(s, lax.bitwise_and(m, jnp.int32(0x7FFFFFFF)))


def _hist_body(feat_ref, sel_ref, out_ref, buf0, buf1, hist, bvec, sem0, sem1,
               *, low_bits):
  wid = lax.axis_index("s") * NC + lax.axis_index("c")

  def _src(ch):
    b_off, rem = divmod(ch, 2 * L)
    lay, half = divmod(rem, 2)
    return feat_ref.at[2 * wid + b_off, lay, pl.ds(half * CHUNK, CHUNK)]

  zero16 = jnp.zeros((16,), jnp.int32)

  def zbody(i, c):
    hist[pl.ds(i * 16, 16)] = zero16
    return c

  lax.fori_loop(0, NBINS // 16, zbody, 0)

  if low_bits:
    pltpu.sync_copy(sel_ref.at[pl.ds(0, 16)], bvec)
    bval = bvec[...]

  bufs = (buf0, buf1)
  sems = (sem0, sem1)
  cps = [None, None]
  cps[0] = pltpu.async_copy(_src(0), buf0, sem0)
  for ch in range(NCH):
    if ch + 1 < NCH:
      cps[(ch + 1) % 2] = pltpu.async_copy(
          _src(ch + 1), bufs[(ch + 1) % 2], sems[(ch + 1) % 2])
    cps[ch % 2].wait()
    buf = bufs[ch % 2]

    @plsc.parallel_loop(0, CHUNK // 16, unroll=UNROLL)
    def _(j):
      v = buf[pl.ds(j * 16, 16)]
      key = _monotonic_key(v)
      hi = lax.shift_right_arithmetic(key, 16) + jnp.int32(32768)
      if low_bits:
        binv = lax.bitwise_and(key, jnp.int32(0xFFFF))
        elig = hi == bval
        cnt, last = plsc.scan_count(binv, elig)
      else:
        binv = hi
        cnt, last = plsc.scan_count(binv)
      plsc.addupdate_scatter(hist, [binv], cnt, mask=last)

  pltpu.sync_copy(hist, out_ref.at[wid])


def _make_hist(low_bits):
  body = functools.partial(_hist_body, low_bits=low_bits)
  return pl.kernel(
      body,
      out_type=jax.ShapeDtypeStruct((NW, NBINS), jnp.int32),
      mesh=_MESH,
      compiler_params=pltpu.CompilerParams(needs_layout_passes=False),
      scratch_types=[
          pltpu.VMEM((CHUNK,), jnp.float32),
          pltpu.VMEM((CHUNK,), jnp.float32),
          pltpu.VMEM((NBINS,), jnp.int32),
          pltpu.VMEM((16,), jnp.int32),
          pltpu.SemaphoreType.DMA,
          pltpu.SemaphoreType.DMA,
      ],
  )


_hist_hi = _make_hist(False)
_hist_lo = _make_hist(True)


def _lin_idx():
  return (lax.broadcasted_iota(jnp.int32, (512, 128), 0) * 128
          + lax.broadcasted_iota(jnp.int32, (512, 128), 1))


def _find_bucket(tot, k):
  lin = _lin_idx()

  def step(_, lohi):
    lo, hi = lohi
    mid = lax.div(lo + hi, jnp.int32(2))
    cnt = jnp.sum(jnp.where(lin > mid, tot, 0))
    ok = cnt < k
    return jnp.where(ok, lo, mid + 1), jnp.where(ok, mid, hi)

  b, _ = lax.fori_loop(0, 16, step, (jnp.int32(0), jnp.int32(NBINS - 1)))
  n_above = jnp.sum(jnp.where(lin > b, tot, 0))
  return b, n_above


def _select1_body(h_ref, out_ref):
  h = h_ref[...]
  tot = jnp.sum(h, axis=0)
  b, n_above = _find_bucket(tot, jnp.int32(BATCH_K))
  r = jnp.int32(BATCH_K) - n_above
  ri = lax.broadcasted_iota(jnp.int32, (8, 128), 0)
  out_ref[...] = jnp.where(ri == 0, b, r)


def _select1(hist3d):
  return pl.pallas_call(
      _select1_body,
      out_shape=jax.ShapeDtypeStruct((8, 128), jnp.int32),
  )(hist3d)


MB_B = 2
_BLIDX = MB_B * L * D


def _mask_body(x_ref, h_ref, sel_ref, o_ref, st_ref, cnt_ref):
  i = pl.program_id(0)

  @pl.when(i == 0)
  def _():
    cnt_ref[0] = 0
    tot = jnp.sum(h_ref[...], axis=0)
    selv = sel_ref[...]
    ri = lax.broadcasted_iota(jnp.int32, (8, 128), 0)
    neg = jnp.int32(-(1 << 30))
    b = jnp.max(jnp.where(ri == 0, selv, neg))
    r = jnp.max(jnp.where(ri == 1, selv, neg))
    tlow, e_at = _find_bucket(tot, r)
    st_ref[0] = lax.shift_left(b - jnp.int32(32768), 16) + tlow
    st_ref[1] = r - e_at

  x = x_ref[...]
  s = lax.bitcast_convert_type(x, jnp.int32)
  m = lax.shift_right_arithmetic(s, 31)
  key = lax.bitwise_xor(s, lax.bitwise_and(m, jnp.int32(0x7FFFFFFF)))
  t = st_ref[0]
  q = st_ref[1]
  gt = key > t
  eq = key == t
  relu = jnp.maximum(x, 0.0)
  zero = jnp.float32(0.0)
  n_eq = jnp.sum(eq.astype(jnp.int32))
  need = q - cnt_ref[0]

  o_ref[...] = jnp.where(gt, relu, zero)

  @pl.when(jnp.logical_and(need > 0, n_eq > 0))
  def _():
    @pl.when(need >= n_eq)
    def _():
      o_ref[...] = jnp.where(jnp.logical_or(gt, eq), relu, zero)

    @pl.when(need < n_eq)
    def _():
      lidx = ((lax.broadcasted_iota(jnp.int32, (MB_B, L, D), 0) * L
               + lax.broadcasted_iota(jnp.int32, (MB_B, L, D), 1)) * D
              + lax.broadcasted_iota(jnp.int32, (MB_B, L, D), 2))

      def step(_, lohi):
        lo, hi = lohi
        mid = lax.div(lo + hi, jnp.int32(2))
        cnt = jnp.sum(jnp.logical_and(eq, lidx <= mid).astype(jnp.int32))
        ok = cnt >= need
        return jnp.where(ok, lo, mid + 1), jnp.where(ok, mid, hi)

      lo, _hi = lax.fori_loop(0, 20, step, (jnp.int32(0),
                                            jnp.int32(_BLIDX - 1)))
      sel = jnp.logical_or(gt, jnp.logical_and(eq, lidx <= lo))
      o_ref[...] = jnp.where(sel, relu, zero)

  cnt_ref[0] = cnt_ref[0] + n_eq


def _mask(x3d, hist3d, sel1):
  return pl.pallas_call(
      _mask_body,
      grid=(B // MB_B,),
      in_specs=[
          pl.BlockSpec((MB_B, L, D), lambda i: (i, 0, 0)),
          pl.BlockSpec((32, 512, 128), lambda i: (0, 0, 0)),
          pl.BlockSpec((8, 128), lambda i: (0, 0)),
      ],
      out_specs=pl.BlockSpec((MB_B, L, D), lambda i: (i, 0, 0)),
      out_shape=jax.ShapeDtypeStruct((B, L, D), jnp.float32),
      scratch_shapes=[pltpu.SMEM((2,), jnp.int32),
                      pltpu.SMEM((1,), jnp.int32)],
  )(x3d, hist3d, sel1)


@jax.jit
def kernel(features):
  hist1 = _hist_hi(features, jnp.zeros((1024,), jnp.int32))
  sel1 = _select1(hist1.reshape(32, 512, 128))
  hist2 = _hist_lo(features, sel1.reshape(-1))
  return _mask(features, hist2.reshape(32, 512, 128), sel1)
